# baseline (speedup 1.0000x reference)
import functools

import jax
import jax.numpy as jnp
from jax import lax
from jax.experimental import pallas as pl
from jax.experimental.pallas import tpu as pltpu
from jax.experimental.pallas import tpu_sc as plsc


@functools.lru_cache(maxsize=None)
def _make_gather(S, P, V, D):
    info = plsc.get_sparse_core_info()
    NC, NS, L = info.num_cores, info.num_subcores, info.num_lanes
    NW = NC * NS

    CT = S // 128
    assert CT % NW == 0
    c_per_w = CT // NW
    KD = D // 8
    NL = 128 // L

    mesh = plsc.VectorSubcoreMesh(core_axis_name="c", subcore_axis_name="s")

    @functools.partial(
        pl.kernel,
        mesh=mesh,
        compiler_params=pltpu.CompilerParams(
            use_tc_tiling_on_sc=False, needs_layout_passes=False
        ),
        out_type=jax.ShapeDtypeStruct((P, KD, CT, 8, 128), jnp.float32),
        scratch_types=[
            pltpu.VMEM((P, 128), jnp.int32),
            pltpu.VMEM((2, 128, D), jnp.float32),
            pltpu.VMEM((2 * D, 129), jnp.float32),
            pltpu.SemaphoreType.DMA((2,)),
            pltpu.SemaphoreType.DMA((2,)),
        ],
    )
    def gather_kernel(table_hbm, post_hbm, out_hbm, idx_v, rows_v, t_v, sem_g, sem_o):
        wid = lax.axis_index("s") * NC + lax.axis_index("c")

        iot = lax.iota(jnp.int32, L)
        rowq = [[iot + (b * D + L * q) for q in range(D // L)] for b in (0, 1)]

        def start_gather(c_, p, b):
            p = min(p, P - 1) if isinstance(p, int) else lax.min(p, P - 1)
            return pltpu.async_copy(
                table_hbm.at[idx_v.at[p]], rows_v.at[b], sem_g.at[b]
            )

        def wait_gather(b):
            pltpu.make_async_copy(
                table_hbm.at[idx_v.at[0]], rows_v.at[b], sem_g.at[b]
            ).wait()

        def transpose(b):
            LB = 4
            for l0 in range(0, 128, LB):
                cols = [jnp.full((L,), l, jnp.int32) for l in range(l0, l0 + LB)]
                vecs = [
                    [rows_v[b, l0 + i, pl.ds(L * q, L)] for q in range(D // L)]
                    for i in range(LB)
                ]
                for i in range(LB):
                    for q in range(D // L):
                        plsc.store_scatter(t_v, [rowq[b][q], cols[i]], vecs[i][q])

        def start_store(c_, p, b):
            for k in range(KD):
                pltpu.async_copy(
                    t_v.at[pl.ds(b * D + 8 * k, 8), pl.ds(0, 128)],
                    out_hbm.at[p, k, c_],
                    sem_o.at[b],
                )

        def wait_store(c_, b):
            for k in range(KD):
                pltpu.make_async_copy(
                    t_v.at[pl.ds(b * D + 8 * k, 8), pl.ds(0, 128)],
                    out_hbm.at[0, k, c_],
                    sem_o.at[b],
                ).wait()

        def c_body(ci, carry):
            c_ = wid * c_per_w + ci
            pltpu.sync_copy(post_hbm.at[:, c_], idx_v)
            start_gather(c_, 0, 0)
            start_gather(c_, 1, 1)

            def body(t, carry2):
                for b in (0, 1):
                    p = 2 * t + b
                    wait_gather(b)

                    @pl.when(t > 0)
                    def _():
                        wait_store(c_, b)

                    transpose(b)
                    start_store(c_, p, b)
                    start_gather(c_, p + 2, b)
                return carry2

            lax.fori_loop(0, P // 2, body, 0)

            for b in (0, 1):
                wait_store(c_, b)
                wait_gather(b)
            return carry

        lax.fori_loop(0, c_per_w, c_body, 0)

    return gather_kernel


def kernel(positions, pe):
    S, P = positions.shape
    V, D = pe.shape
    gather = _make_gather(S, P, V, D)
    post = positions.T.reshape(P, S // 128, 128).astype(jnp.int32)
    x = gather(pe.astype(jnp.float32), post)
    return x.transpose(2, 4, 0, 1, 3).reshape(S, P, D)


# speedup vs baseline: 1.0095x; 1.0095x over previous
import functools

import jax
import jax.numpy as jnp
from jax import lax
from jax.experimental import pallas as pl
from jax.experimental.pallas import tpu as pltpu
from jax.experimental.pallas import tpu_sc as plsc


@functools.lru_cache(maxsize=None)
def _make_gather(S, P, V, D):
    info = plsc.get_sparse_core_info()
    NC, NS, L = info.num_cores, info.num_subcores, info.num_lanes
    NW = NC * NS

    CT = S // 128
    assert CT % NW == 0
    c_per_w = CT // NW
    KD = D // 8
    NL = 128 // L

    mesh = plsc.VectorSubcoreMesh(core_axis_name="c", subcore_axis_name="s")

    @functools.partial(
        pl.kernel,
        mesh=mesh,
        compiler_params=pltpu.CompilerParams(
            use_tc_tiling_on_sc=False, needs_layout_passes=False
        ),
        out_type=jax.ShapeDtypeStruct((P, KD, CT, 8, 128), jnp.float32),
        scratch_types=[
            pltpu.VMEM((P, 128), jnp.int32),
            pltpu.VMEM((2, 128, D), jnp.float32),
            pltpu.VMEM((2 * D, 129), jnp.float32),
            pltpu.SemaphoreType.DMA((2,)),
            pltpu.SemaphoreType.DMA((2,)),
        ],
    )
    def gather_kernel(table_hbm, post_hbm, out_hbm, idx_v, rows_v, t_v, sem_g, sem_o):
        wid = lax.axis_index("s") * NC + lax.axis_index("c")

        iot = lax.iota(jnp.int32, L)
        rowq = [[iot + (b * D + L * q) for q in range(D // L)] for b in (0, 1)]

        def start_gather(c_, p, b):
            p = min(p, P - 1) if isinstance(p, int) else lax.min(p, P - 1)
            return pltpu.async_copy(
                table_hbm.at[idx_v.at[p]], rows_v.at[b], sem_g.at[b]
            )

        def wait_gather(b):
            pltpu.make_async_copy(
                table_hbm.at[idx_v.at[0]], rows_v.at[b], sem_g.at[b]
            ).wait()

        def transpose(b):
            LB = 2
            for l0 in range(0, 128, LB):
                cols = [jnp.full((L,), l, jnp.int32) for l in range(l0, l0 + LB)]
                vecs = [
                    [rows_v[b, l0 + i, pl.ds(L * q, L)] for q in range(D // L)]
                    for i in range(LB)
                ]
                for i in range(LB):
                    for q in range(D // L):
                        plsc.store_scatter(t_v, [rowq[b][q], cols[i]], vecs[i][q])

        def start_store(c_, p, b):
            for k in range(KD):
                pltpu.async_copy(
                    t_v.at[pl.ds(b * D + 8 * k, 8), pl.ds(0, 128)],
                    out_hbm.at[p, k, c_],
                    sem_o.at[b],
                )

        def wait_store(c_, b):
            for k in range(KD):
                pltpu.make_async_copy(
                    t_v.at[pl.ds(b * D + 8 * k, 8), pl.ds(0, 128)],
                    out_hbm.at[0, k, c_],
                    sem_o.at[b],
                ).wait()

        def c_body(ci, carry):
            c_ = wid * c_per_w + ci
            pltpu.sync_copy(post_hbm.at[:, c_], idx_v)
            start_gather(c_, 0, 0)
            start_gather(c_, 1, 1)

            def body(t, carry2):
                for b in (0, 1):
                    p = 2 * t + b
                    wait_gather(b)

                    @pl.when(t > 0)
                    def _():
                        wait_store(c_, b)

                    transpose(b)
                    start_store(c_, p, b)
                    start_gather(c_, p + 2, b)
                return carry2

            lax.fori_loop(0, P // 2, body, 0)

            for b in (0, 1):
                wait_store(c_, b)
                wait_gather(b)
            return carry

        lax.fori_loop(0, c_per_w, c_body, 0)

    return gather_kernel


def kernel(positions, pe):
    S, P = positions.shape
    V, D = pe.shape
    gather = _make_gather(S, P, V, D)
    post = positions.T.reshape(P, S // 128, 128).astype(jnp.int32)
    x = gather(pe.astype(jnp.float32), post)
    return x.transpose(2, 4, 0, 1, 3).reshape(S, P, D)
